# Initial kernel scaffold; baseline (speedup 1.0000x reference)
import jax
import jax.numpy as jnp
from jax.experimental import pallas as pl


def kernel(logits, targets):
    raise NotImplementedError("write your pallas kernel here")


# speedup vs baseline: 41.8304x; 41.8304x over previous
import functools

import jax
import jax.numpy as jnp
from jax import lax
from jax.experimental import pallas as pl
from jax.experimental.pallas import tpu as pltpu
from jax.experimental.pallas import tpu_sc as plsc

IGNORE = 255
NCLS = 19
K = 4096
CK = NCLS * K
NSLOT = 2 * CK + 128
LC = 8192
BURST_ROWS = 16


def _stage1_body(logits_ref, tgt_ref, idx_ref, fg_ref):
    x = logits_ref[0]
    t = tgt_ref[0, 0]
    m = jnp.max(x, axis=0, keepdims=True)
    e = jnp.exp(x - m)
    p = e / jnp.sum(e, axis=0, keepdims=True)
    cls = lax.broadcasted_iota(jnp.int32, (NCLS, LC), 0)
    valid = t != IGNORE
    fg = (t == cls) & valid
    err = jnp.where(fg, p, 1.0 - p) * valid.astype(jnp.float32)
    b = jnp.clip((err * (K - 1) + 0.5).astype(jnp.int32), 0, K - 1)
    rb = (K - 1) - b
    idx_ref[0] = cls * K + rb
    rb_fg = jnp.sum(jnp.where(fg, rb, 0), axis=0, keepdims=True)
    cls_fg = jnp.sum(jnp.where(fg, cls, 0), axis=0, keepdims=True)
    fg_ref[0, 0] = jnp.where(valid, CK + cls_fg * K + rb_fg, 2 * CK)


def _stage1(logits3, targets4, B, HW):
    grid = (B, HW // LC)
    return pl.pallas_call(
        _stage1_body,
        grid=grid,
        in_specs=[
            pl.BlockSpec((1, NCLS, LC), lambda b, j: (b, 0, j)),
            pl.BlockSpec((1, 1, 1, LC), lambda b, j: (b, j, 0, 0)),
        ],
        out_specs=[
            pl.BlockSpec((1, NCLS, LC), lambda b, j: (b, 0, j)),
            pl.BlockSpec((1, 1, 1, LC), lambda b, j: (b, j, 0, 0)),
        ],
        out_shape=[
            jax.ShapeDtypeStruct((B, NCLS, HW), jnp.int32),
            jax.ShapeDtypeStruct((B, HW // LC, 1, LC), jnp.int32),
        ],
    )(logits3, targets4)


def _make_stage2(n_rows, f_rows):
    mesh = plsc.VectorSubcoreMesh(core_axis_name="c", subcore_axis_name="s")
    nw = 32
    n_per_w = n_rows // nw
    f_per_w = f_rows // nw
    zch = NSLOT // 16

    @functools.partial(
        pl.kernel,
        out_type=jax.ShapeDtypeStruct((2 * NSLOT,), jnp.float32),
        mesh=mesh,
        scratch_types=[
            pltpu.VMEM((BURST_ROWS, 128), jnp.int32),
            pltpu.VMEM((128,), jnp.float32),
            pltpu.VMEM((zch,), jnp.float32),
            pltpu.VMEM_SHARED((NSLOT,), jnp.float32),
            pltpu.SemaphoreType.DMA,
        ],
    )
    def hist_kernel(idxn_hbm, idxf_hbm, ones_hbm, zeros_hbm, out_hbm,
                    idx_v, ones_v, stage_v, hist_sh, sem):
        cid = lax.axis_index("c")
        sid = lax.axis_index("s")
        wid = cid * 16 + sid
        pltpu.sync_copy(zeros_hbm, stage_v)
        pltpu.sync_copy(stage_v, hist_sh.at[pl.ds(sid * zch, zch)])
        pltpu.sync_copy(ones_hbm, ones_v)
        plsc.subcore_barrier()

        def scatter_range(src_hbm, base_rows, num_bursts):
            def body(g, carry):
                row0 = base_rows + g * BURST_ROWS
                pltpu.sync_copy(src_hbm.at[pl.ds(row0, BURST_ROWS)], idx_v)
                descs = [
                    pltpu.async_copy(ones_v, hist_sh.at[idx_v.at[j]], sem,
                                     add=True)
                    for j in range(BURST_ROWS)
                ]
                for d in descs:
                    d.wait()
                return carry
            lax.fori_loop(0, num_bursts, body, 0)

        scatter_range(idxn_hbm, wid * n_per_w, n_per_w // BURST_ROWS)
        scatter_range(idxf_hbm, wid * f_per_w, f_per_w // BURST_ROWS)
        plsc.subcore_barrier()
        pltpu.sync_copy(hist_sh.at[pl.ds(sid * zch, zch)], stage_v)
        pltpu.sync_copy(stage_v,
                        out_hbm.at[pl.ds(cid * NSLOT + sid * zch, zch)])

    return hist_kernel


ROWS = NCLS * (K // 128)
RPC = K // 128


def _finish_body(nh_ref, fh_ref, out_ref):
    n = nh_ref[0] + nh_ref[1]
    f = fh_ref[0] + fh_ref[1]
    mm = lambda a, b: lax.dot_general(
        a, b, (((1,), (0,)), ((), ())),
        preferred_element_type=jnp.float32)
    ia = lax.broadcasted_iota(jnp.int32, (128, 128), 0)
    ja = lax.broadcasted_iota(jnp.int32, (128, 128), 1)
    tri = (ia <= ja).astype(jnp.float32)
    nrc = mm(n, tri)
    frc = mm(f, tri)
    ib = lax.broadcasted_iota(jnp.int32, (ROWS, ROWS), 0)
    jb = lax.broadcasted_iota(jnp.int32, (ROWS, ROWS), 1)
    same = ib // RPC == jb // RPC
    pre = (same & (jb < ib)).astype(jnp.float32)
    tot = same.astype(jnp.float32)
    rt = jnp.concatenate([nrc[:, 127:128], frc[:, 127:128]], axis=1)
    pref = mm(pre, rt)
    icum = nrc + pref[:, 0:1]
    fcum = frc + pref[:, 1:2]
    G = mm(tot, rt[:, 1:2])
    J = 1.0 - (G - fcum) / jnp.maximum(G + icum - fcum, 1.0)
    present = (G > 0).astype(jnp.float32)
    tot_j = jnp.sum(J * present)
    npres = jnp.sum(present) * (1.0 / RPC)
    loss = ((tot_j - npres) / (K - 1)) / jnp.maximum(npres, 1.0)
    out_ref[...] = jnp.full((1, 1), loss, jnp.float32)


def _finish(nh, fh):
    return pl.pallas_call(
        _finish_body,
        out_shape=jax.ShapeDtypeStruct((1, 1), jnp.float32),
    )(nh, fh)


def kernel(logits, targets):
    B, C, H, W = logits.shape
    HW = H * W
    logits3 = logits.astype(jnp.float32).reshape(B, C, HW)
    targets4 = targets.astype(jnp.int32).reshape(B, HW // LC, 1, LC)
    idxn, idxf = _stage1(logits3, targets4, B, HW)
    idxn2 = idxn.reshape(-1, 128)
    idxf2 = idxf.reshape(-1, 128)
    ones = jnp.ones((128,), jnp.float32)
    zeros = jnp.zeros((NSLOT // 16,), jnp.float32)
    hist = _make_stage2(idxn2.shape[0], idxf2.shape[0])(
        idxn2, idxf2, ones, zeros).reshape(2, NSLOT)
    nh = hist[:, :CK].reshape(2, ROWS, 128)
    fh = hist[:, CK:2 * CK].reshape(2, ROWS, 128)
    return _finish(nh, fh).reshape(())


# speedup vs baseline: 59.8644x; 1.4311x over previous
import functools

import jax
import jax.numpy as jnp
from jax import lax
from jax.experimental import pallas as pl
from jax.experimental.pallas import tpu as pltpu
from jax.experimental.pallas import tpu_sc as plsc

IGNORE = 255
NCLS = 19
K = 4096
CK = NCLS * K
NSLOT = 2 * CK + 128
LC = 8192
BURST_ROWS = 16


HB = 64


def _stage1_body(logits_ref, tgt_ref, idx_ref, fg_ref):
    x = logits_ref[0]
    t = tgt_ref[0]
    m = jnp.max(x, axis=0)
    e = jnp.exp(x - m)
    r = (K - 1.0) / jnp.sum(e, axis=0)
    valid = t != IGNORE
    fg_acc = jnp.zeros(t.shape, jnp.int32)
    for c in range(NCLS):
        q = jnp.clip((e[c] * r + 0.5).astype(jnp.int32), 0, K - 1)
        fgc = (t == c) & valid
        rb = jnp.where(fgc, (K - 1) - q, jnp.where(valid, q, K - 1))
        idx_ref[c * HB:(c + 1) * HB, :] = rb + (c * K)
        fg_acc = fg_acc + jnp.where(fgc, (CK + c * K + (K - 1)) - q, 0)
    fg_ref[...] = jnp.where(valid, fg_acc, 2 * CK)


def _stage1(logits, targets, B, H, W):
    grid = (B, H // HB, W // 128)
    nsteps = B * (H // HB) * (W // 128)
    rmap = lambda b, h, w: (b * ((H // HB) * (W // 128)) + h * (W // 128) + w, 0)
    return pl.pallas_call(
        _stage1_body,
        grid=grid,
        in_specs=[
            pl.BlockSpec((1, NCLS, HB, 128), lambda b, h, w: (b, 0, h, w)),
            pl.BlockSpec((1, HB, 128), lambda b, h, w: (b, h, w)),
        ],
        out_specs=[
            pl.BlockSpec((NCLS * HB, 128), rmap),
            pl.BlockSpec((HB, 128), rmap),
        ],
        out_shape=[
            jax.ShapeDtypeStruct((nsteps * NCLS * HB, 128), jnp.int32),
            jax.ShapeDtypeStruct((nsteps * HB, 128), jnp.int32),
        ],
    )(logits, targets)


def _make_stage2(n_rows, f_rows):
    mesh = plsc.VectorSubcoreMesh(core_axis_name="c", subcore_axis_name="s")
    nw = 32
    n_per_w = n_rows // nw
    f_per_w = f_rows // nw
    zch = NSLOT // 16

    @functools.partial(
        pl.kernel,
        out_type=jax.ShapeDtypeStruct((2 * NSLOT,), jnp.float32),
        mesh=mesh,
        scratch_types=[
            pltpu.VMEM((BURST_ROWS, 128), jnp.int32),
            pltpu.VMEM((128,), jnp.float32),
            pltpu.VMEM((zch,), jnp.float32),
            pltpu.VMEM_SHARED((NSLOT,), jnp.float32),
            pltpu.SemaphoreType.DMA,
        ],
    )
    def hist_kernel(idxn_hbm, idxf_hbm, ones_hbm, zeros_hbm, out_hbm,
                    idx_v, ones_v, stage_v, hist_sh, sem):
        cid = lax.axis_index("c")
        sid = lax.axis_index("s")
        wid = cid * 16 + sid
        pltpu.sync_copy(zeros_hbm, stage_v)
        pltpu.sync_copy(stage_v, hist_sh.at[pl.ds(sid * zch, zch)])
        pltpu.sync_copy(ones_hbm, ones_v)
        plsc.subcore_barrier()

        def scatter_range(src_hbm, base_rows, num_bursts):
            def body(g, carry):
                row0 = base_rows + g * BURST_ROWS
                pltpu.sync_copy(src_hbm.at[pl.ds(row0, BURST_ROWS)], idx_v)
                descs = [
                    pltpu.async_copy(ones_v, hist_sh.at[idx_v.at[j]], sem,
                                     add=True)
                    for j in range(BURST_ROWS)
                ]
                for d in descs:
                    d.wait()
                return carry
            lax.fori_loop(0, num_bursts, body, 0)

        scatter_range(idxn_hbm, wid * n_per_w, n_per_w // BURST_ROWS)
        scatter_range(idxf_hbm, wid * f_per_w, f_per_w // BURST_ROWS)
        plsc.subcore_barrier()
        pltpu.sync_copy(hist_sh.at[pl.ds(sid * zch, zch)], stage_v)
        pltpu.sync_copy(stage_v,
                        out_hbm.at[pl.ds(cid * NSLOT + sid * zch, zch)])

    return hist_kernel


ROWS = NCLS * (K // 128)
RPC = K // 128


def _finish_body(nh_ref, fh_ref, out_ref):
    n = nh_ref[0] + nh_ref[1]
    f = fh_ref[0] + fh_ref[1]
    mm = lambda a, b: lax.dot_general(
        a, b, (((1,), (0,)), ((), ())),
        preferred_element_type=jnp.float32)
    ia = lax.broadcasted_iota(jnp.int32, (128, 128), 0)
    ja = lax.broadcasted_iota(jnp.int32, (128, 128), 1)
    tri = (ia <= ja).astype(jnp.float32)
    nrc = mm(n, tri)
    frc = mm(f, tri)
    ib = lax.broadcasted_iota(jnp.int32, (ROWS, ROWS), 0)
    jb = lax.broadcasted_iota(jnp.int32, (ROWS, ROWS), 1)
    same = ib // RPC == jb // RPC
    pre = (same & (jb < ib)).astype(jnp.float32)
    tot = same.astype(jnp.float32)
    rt = jnp.concatenate([nrc[:, 127:128], frc[:, 127:128]], axis=1)
    pref = mm(pre, rt)
    icum = nrc + pref[:, 0:1]
    fcum = frc + pref[:, 1:2]
    G = mm(tot, rt[:, 1:2])
    J = 1.0 - (G - fcum) / jnp.maximum(G + icum - fcum, 1.0)
    present = (G > 0).astype(jnp.float32)
    tot_j = jnp.sum(J * present)
    npres = jnp.sum(present) * (1.0 / RPC)
    loss = ((tot_j - npres) / (K - 1)) / jnp.maximum(npres, 1.0)
    out_ref[...] = jnp.full((1, 1), loss, jnp.float32)


def _finish(nh, fh):
    return pl.pallas_call(
        _finish_body,
        out_shape=jax.ShapeDtypeStruct((1, 1), jnp.float32),
    )(nh, fh)


def kernel(logits, targets):
    B, C, H, W = logits.shape
    idxn2, idxf2 = _stage1(logits.astype(jnp.float32),
                           targets.astype(jnp.int32), B, H, W)
    ones = jnp.ones((128,), jnp.float32)
    zeros = jnp.zeros((NSLOT // 16,), jnp.float32)
    hist = _make_stage2(idxn2.shape[0], idxf2.shape[0])(
        idxn2, idxf2, ones, zeros).reshape(2, NSLOT)
    nh = hist[:, :CK].reshape(2, ROWS, 128)
    fh = hist[:, CK:2 * CK].reshape(2, ROWS, 128)
    return _finish(nh, fh).reshape(())


# speedup vs baseline: 80.7070x; 1.3482x over previous
import functools

import jax
import jax.numpy as jnp
from jax import lax
from jax.experimental import pallas as pl
from jax.experimental.pallas import tpu as pltpu
from jax.experimental.pallas import tpu_sc as plsc

IGNORE = 255
NCLS = 19
K = 2048
CK = NCLS * K
NSLOT = 2 * CK + 128
HB = 64
NW = 32
CHUNK = 8192


def _stage1_body(logits_ref, tgt_ref, idx_ref, fg_ref):
    x = logits_ref[0]
    t = tgt_ref[0]
    m = jnp.max(x, axis=0)
    e = jnp.exp(x - m)
    r = (K - 1.0) / jnp.sum(e, axis=0)
    valid = t != IGNORE
    fg_acc = jnp.zeros(t.shape, jnp.int32)
    for c in range(NCLS):
        q = jnp.clip((e[c] * r + 0.5).astype(jnp.int32), 0, K - 1)
        fgc = (t == c) & valid
        rb = jnp.where(fgc, (K - 1) - q, jnp.where(valid, q, K - 1))
        idx_ref[c * HB:(c + 1) * HB, :] = rb + (c * K)
        fg_acc = fg_acc + jnp.where(fgc, (CK + c * K + (K - 1)) - q, 0)
    fg_ref[...] = jnp.where(valid, fg_acc, 2 * CK)


def _stage1(logits, targets, B, H, W):
    grid = (B, H // HB, W // 128)
    nsteps = B * (H // HB) * (W // 128)
    rmap = lambda b, h, w: (b * ((H // HB) * (W // 128)) + h * (W // 128) + w, 0)
    return pl.pallas_call(
        _stage1_body,
        grid=grid,
        in_specs=[
            pl.BlockSpec((1, NCLS, HB, 128), lambda b, h, w: (b, 0, h, w)),
            pl.BlockSpec((1, HB, 128), lambda b, h, w: (b, h, w)),
        ],
        out_specs=[
            pl.BlockSpec((NCLS * HB, 128), rmap),
            pl.BlockSpec((HB, 128), rmap),
        ],
        out_shape=[
            jax.ShapeDtypeStruct((nsteps * NCLS * HB, 128), jnp.int32),
            jax.ShapeDtypeStruct((nsteps * HB, 128), jnp.int32),
        ],
    )(logits, targets)


CROWS = CHUNK // 128


def _make_stage2(n_rows, f_rows):
    mesh = plsc.VectorSubcoreMesh(core_axis_name="c", subcore_axis_name="s")
    n_per_w = n_rows // NW
    f_per_w = f_rows // NW
    n_chunks = n_per_w // CROWS
    f_chunks = f_per_w // CROWS

    @functools.partial(
        pl.kernel,
        out_type=jax.ShapeDtypeStruct((NW * NSLOT,), jnp.int32),
        mesh=mesh,
        compiler_params=pltpu.CompilerParams(needs_layout_passes=False),
        scratch_types=[
            pltpu.VMEM((CROWS, 128), jnp.int32),
            pltpu.VMEM((CROWS, 128), jnp.int32),
            pltpu.VMEM((NSLOT,), jnp.int32),
            pltpu.SemaphoreType.DMA,
            pltpu.SemaphoreType.DMA,
        ],
    )
    def hist_kernel(idxn_hbm, idxf_hbm, zeros_hbm, out_hbm,
                    buf0, buf1, hist_v, sem0, sem1):
        cid = lax.axis_index("c")
        sid = lax.axis_index("s")
        wid = cid * 16 + sid
        pltpu.sync_copy(zeros_hbm, hist_v)
        bufs = (buf0, buf1)
        sems = (sem0, sem1)

        ones16 = jnp.ones((16,), jnp.int32)

        def process(buf):
            def row_body(r, carry):
                for k in range(8):
                    x = buf[r, pl.ds(k * 16, 16)]
                    plsc.addupdate_scatter(hist_v, [x], ones16)
                return carry
            lax.fori_loop(0, CROWS, row_body, 0)

        def scatter_range(src_hbm, base_row, num_chunks):
            for b in range(2):
                pltpu.async_copy(
                    src_hbm.at[pl.ds(base_row + b * CROWS, CROWS)],
                    bufs[b], sems[b])

            def pair_body(i, carry):
                for b in range(2):
                    pltpu.make_async_copy(
                        src_hbm.at[pl.ds(base_row, CROWS)],
                        bufs[b], sems[b]).wait()
                    process(bufs[b])
                    nxt = (2 * i + 2 + b) * CROWS
                    pltpu.async_copy(
                        src_hbm.at[pl.ds(base_row + nxt, CROWS)],
                        bufs[b], sems[b])
                return carry
            lax.fori_loop(0, num_chunks // 2 - 1, pair_body, 0)
            for b in range(2):
                pltpu.make_async_copy(
                    src_hbm.at[pl.ds(base_row, CROWS)],
                    bufs[b], sems[b]).wait()
                process(bufs[b])

        scatter_range(idxn_hbm, wid * n_per_w, n_chunks)
        scatter_range(idxf_hbm, wid * f_per_w, f_chunks)
        pltpu.sync_copy(hist_v, out_hbm.at[pl.ds(wid * NSLOT, NSLOT)])

    return hist_kernel


ROWS = NCLS * (K // 128)
RPC = K // 128


def _finish_body(nh_ref, fh_ref, out_ref):
    n = nh_ref[0]
    f = fh_ref[0]
    for w in range(1, NW):
        n = n + nh_ref[w]
        f = f + fh_ref[w]
    n = n.astype(jnp.float32)
    f = f.astype(jnp.float32)
    mm = lambda a, b: lax.dot_general(
        a, b, (((1,), (0,)), ((), ())),
        preferred_element_type=jnp.float32)
    ia = lax.broadcasted_iota(jnp.int32, (128, 128), 0)
    ja = lax.broadcasted_iota(jnp.int32, (128, 128), 1)
    tri = (ia <= ja).astype(jnp.float32)
    nrc = mm(n, tri)
    frc = mm(f, tri)
    ib = lax.broadcasted_iota(jnp.int32, (ROWS, ROWS), 0)
    jb = lax.broadcasted_iota(jnp.int32, (ROWS, ROWS), 1)
    same = ib // RPC == jb // RPC
    pre = (same & (jb < ib)).astype(jnp.float32)
    tot = same.astype(jnp.float32)
    rt = jnp.concatenate([nrc[:, 127:128], frc[:, 127:128]], axis=1)
    pref = mm(pre, rt)
    icum = nrc + pref[:, 0:1]
    fcum = frc + pref[:, 1:2]
    G = mm(tot, rt[:, 1:2])
    J = 1.0 - (G - fcum) / jnp.maximum(G + icum - fcum, 1.0)
    present = (G > 0).astype(jnp.float32)
    tot_j = jnp.sum(J * present)
    npres = jnp.sum(present) * (1.0 / RPC)
    loss = ((tot_j - npres) / (K - 1)) / jnp.maximum(npres, 1.0)
    out_ref[...] = jnp.full((1, 1), loss, jnp.float32)


def _finish(nh, fh):
    return pl.pallas_call(
        _finish_body,
        out_shape=jax.ShapeDtypeStruct((1, 1), jnp.float32),
    )(nh, fh)


def kernel(logits, targets):
    B, C, H, W = logits.shape
    idxn2, idxf2 = _stage1(logits.astype(jnp.float32),
                           targets.astype(jnp.int32), B, H, W)
    zeros = jnp.zeros((NSLOT,), jnp.int32)
    hist = _make_stage2(idxn2.shape[0], idxf2.shape[0])(
        idxn2, idxf2, zeros).reshape(NW, NSLOT)
    nh = hist[:, :CK].reshape(NW, ROWS, 128)
    fh = hist[:, CK:2 * CK].reshape(NW, ROWS, 128)
    return _finish(nh, fh).reshape(())


# speedup vs baseline: 155.0796x; 1.9215x over previous
import functools

import jax
import jax.numpy as jnp
from jax import lax
from jax.experimental import pallas as pl
from jax.experimental.pallas import tpu as pltpu
from jax.experimental.pallas import tpu_sc as plsc

IGNORE = 255
NCLS = 19
K = 2048
CK = NCLS * K
NSLOT = 2 * CK + 128
HB = 64
NW = 32
CHUNK = 8192


def _stage1_body(logits_ref, tgt_ref, idx_ref, fg_ref):
    x = logits_ref[0]
    t = tgt_ref[0]
    m = jnp.max(x, axis=0)
    e = jnp.exp(x - m)
    r = (K - 1.0) / jnp.sum(e, axis=0)
    valid = t != IGNORE
    fg_acc = jnp.zeros(t.shape, jnp.int32)
    for c in range(NCLS):
        q = jnp.clip((e[c] * r + 0.5).astype(jnp.int32), 0, K - 1)
        fgc = (t == c) & valid
        rb = jnp.where(fgc, (K - 1) - q, jnp.where(valid, q, K - 1))
        full = rb + (c * K)
        idx_ref[c * (HB // 2):(c + 1) * (HB // 2), :] = (
            full[:HB // 2] | (full[HB // 2:] << 16))
        fg_acc = fg_acc + jnp.where(fgc, (CK + c * K + (K - 1)) - q, 0)
    fg_ref[...] = jnp.where(valid, fg_acc, 2 * CK)


def _stage1(logits, targets, B, H, W):
    grid = (B, H // HB, W // 128)
    nsteps = B * (H // HB) * (W // 128)
    rmap = lambda b, h, w: (b * ((H // HB) * (W // 128)) + h * (W // 128) + w, 0)
    return pl.pallas_call(
        _stage1_body,
        grid=grid,
        in_specs=[
            pl.BlockSpec((1, NCLS, HB, 128), lambda b, h, w: (b, 0, h, w)),
            pl.BlockSpec((1, HB, 128), lambda b, h, w: (b, h, w)),
        ],
        out_specs=[
            pl.BlockSpec((NCLS * (HB // 2), 128), rmap),
            pl.BlockSpec((HB, 128), rmap),
        ],
        out_shape=[
            jax.ShapeDtypeStruct((nsteps * NCLS * (HB // 2), 128), jnp.int32),
            jax.ShapeDtypeStruct((nsteps * HB, 128), jnp.int32),
        ],
    )(logits, targets)


CROWS = CHUNK // 128


def _make_stage2(n_rows, f_rows):
    mesh = plsc.VectorSubcoreMesh(core_axis_name="c", subcore_axis_name="s")
    n_per_w = n_rows // NW
    f_per_w = f_rows // NW
    n_chunks = n_per_w // CROWS
    f_chunks = f_per_w // CROWS

    @functools.partial(
        pl.kernel,
        out_type=jax.ShapeDtypeStruct((NW * NSLOT,), jnp.int32),
        mesh=mesh,
        compiler_params=pltpu.CompilerParams(needs_layout_passes=False),
        scratch_types=[
            pltpu.VMEM((CROWS, 128), jnp.int32),
            pltpu.VMEM((CROWS, 128), jnp.int32),
            pltpu.VMEM((NSLOT,), jnp.int32),
            pltpu.SemaphoreType.DMA,
            pltpu.SemaphoreType.DMA,
        ],
    )
    def hist_kernel(idxn_hbm, idxf_hbm, zeros_hbm, out_hbm,
                    buf0, buf1, hist_v, sem0, sem1):
        cid = lax.axis_index("c")
        sid = lax.axis_index("s")
        wid = cid * 16 + sid
        pltpu.sync_copy(zeros_hbm, hist_v)
        bufs = (buf0, buf1)
        sems = (sem0, sem1)

        ones16 = jnp.ones((16,), jnp.int32)

        def process_packed(buf):
            def row_body(r, carry):
                xs = [buf[r, pl.ds(k * 16, 16)] for k in range(8)]
                los = [x & 0xFFFF for x in xs]
                his = [lax.shift_right_logical(x, 16) for x in xs]
                for v in los + his:
                    plsc.addupdate_scatter(hist_v, [v], ones16)
                return carry
            lax.fori_loop(0, CROWS, row_body, 0)

        def process_raw(buf):
            def row_body(r, carry):
                xs = [buf[r, pl.ds(k * 16, 16)] for k in range(8)]
                for x in xs:
                    plsc.addupdate_scatter(hist_v, [x], ones16)
                return carry
            lax.fori_loop(0, CROWS, row_body, 0)

        def scatter_range(src_hbm, base_row, num_chunks, process):
            for b in range(2):
                pltpu.async_copy(
                    src_hbm.at[pl.ds(base_row + b * CROWS, CROWS)],
                    bufs[b], sems[b])

            def pair_body(i, carry):
                for b in range(2):
                    pltpu.make_async_copy(
                        src_hbm.at[pl.ds(base_row, CROWS)],
                        bufs[b], sems[b]).wait()
                    process(bufs[b])
                    nxt = (2 * i + 2 + b) * CROWS
                    pltpu.async_copy(
                        src_hbm.at[pl.ds(base_row + nxt, CROWS)],
                        bufs[b], sems[b])
                return carry
            lax.fori_loop(0, num_chunks // 2 - 1, pair_body, 0)
            for b in range(2):
                pltpu.make_async_copy(
                    src_hbm.at[pl.ds(base_row, CROWS)],
                    bufs[b], sems[b]).wait()
                process(bufs[b])

        scatter_range(idxn_hbm, wid * n_per_w, n_chunks, process_packed)
        scatter_range(idxf_hbm, wid * f_per_w, f_chunks, process_raw)
        pltpu.sync_copy(hist_v, out_hbm.at[pl.ds(wid * NSLOT, NSLOT)])

    return hist_kernel


ROWS = NCLS * (K // 128)
RPC = K // 128


def _finish_body(nh_ref, fh_ref, out_ref):
    n = nh_ref[0]
    f = fh_ref[0]
    for w in range(1, nh_ref.shape[0]):
        n = n + nh_ref[w]
        f = f + fh_ref[w]
    n = n.astype(jnp.float32)
    f = f.astype(jnp.float32)
    mm = lambda a, b: lax.dot_general(
        a, b, (((1,), (0,)), ((), ())),
        preferred_element_type=jnp.float32)
    ia = lax.broadcasted_iota(jnp.int32, (128, 128), 0)
    ja = lax.broadcasted_iota(jnp.int32, (128, 128), 1)
    tri = (ia <= ja).astype(jnp.float32)
    nrc = mm(n, tri)
    frc = mm(f, tri)
    ib = lax.broadcasted_iota(jnp.int32, (ROWS, ROWS), 0)
    jb = lax.broadcasted_iota(jnp.int32, (ROWS, ROWS), 1)
    same = ib // RPC == jb // RPC
    pre = (same & (jb < ib)).astype(jnp.float32)
    tot = same.astype(jnp.float32)
    rt = jnp.concatenate([nrc[:, 127:128], frc[:, 127:128]], axis=1)
    pref = mm(pre, rt)
    icum = nrc + pref[:, 0:1]
    fcum = frc + pref[:, 1:2]
    G = mm(tot, rt[:, 1:2])
    J = 1.0 - (G - fcum) / jnp.maximum(G + icum - fcum, 1.0)
    present = (G > 0).astype(jnp.float32)
    tot_j = jnp.sum(J * present)
    npres = jnp.sum(present) * (1.0 / RPC)
    loss = ((tot_j - npres) / (K - 1)) / jnp.maximum(npres, 1.0)
    out_ref[...] = jnp.full((1, 1), loss, jnp.float32)


def _finish(nh, fh):
    return pl.pallas_call(
        _finish_body,
        out_shape=jax.ShapeDtypeStruct((1, 1), jnp.float32),
    )(nh, fh)


def kernel(logits, targets):
    B, C, H, W = logits.shape
    idxn2, idxf2 = _stage1(logits.astype(jnp.float32),
                           targets.astype(jnp.int32), B, H, W)
    zeros = jnp.zeros((NSLOT,), jnp.int32)
    hist = _make_stage2(idxn2.shape[0], idxf2.shape[0])(
        idxn2, idxf2, zeros).reshape(NW, NSLOT)
    nh = hist[:, :CK].reshape(NW, ROWS, 128)
    fh = hist[:, CK:2 * CK].reshape(NW, ROWS, 128)
    return _finish(nh, fh).reshape(())


# speedup vs baseline: 182.3877x; 1.1761x over previous
import functools

import jax
import jax.numpy as jnp
from jax import lax
from jax.experimental import pallas as pl
from jax.experimental.pallas import tpu as pltpu
from jax.experimental.pallas import tpu_sc as plsc

IGNORE = 255
NCLS = 19
K = 2048
CK = NCLS * K
NSLOT = 2 * CK + 128
HB = 128
NW = 32
CHUNK = 8192


def _stage1_body(logits_ref, tgt_ref, idx_ref, fg_ref):
    x = logits_ref[0]
    t = tgt_ref[0]
    m = jnp.max(x, axis=0)
    e = jnp.exp(x - m)
    r = (K - 1.0) / jnp.sum(e, axis=0)
    valid = t != IGNORE
    fg_acc = jnp.zeros(t.shape, jnp.int32)
    for c in range(NCLS):
        q = jnp.clip((e[c] * r + 0.5).astype(jnp.int32), 0, K - 1)
        fgc = (t == c) & valid
        rb = jnp.where(fgc, (K - 1) - q, jnp.where(valid, q, K - 1))
        full = rb + (c * K)
        idx_ref[c * (HB // 2):(c + 1) * (HB // 2), :] = (
            full[:HB // 2] | (full[HB // 2:] << 16))
        fg_acc = fg_acc + jnp.where(fgc, (CK + c * K + (K - 1)) - q, 0)
    fg_ref[...] = jnp.where(valid, fg_acc, 2 * CK)


def _stage1(logits, targets, B, H, W):
    grid = (B, H // HB, W // 128)
    nsteps = B * (H // HB) * (W // 128)
    rmap = lambda b, h, w: (b * ((H // HB) * (W // 128)) + h * (W // 128) + w, 0)
    return pl.pallas_call(
        _stage1_body,
        grid=grid,
        in_specs=[
            pl.BlockSpec((1, NCLS, HB, 128), lambda b, h, w: (b, 0, h, w)),
            pl.BlockSpec((1, HB, 128), lambda b, h, w: (b, h, w)),
        ],
        out_specs=[
            pl.BlockSpec((NCLS * (HB // 2), 128), rmap),
            pl.BlockSpec((HB, 128), rmap),
        ],
        out_shape=[
            jax.ShapeDtypeStruct((nsteps * NCLS * (HB // 2), 128), jnp.int32),
            jax.ShapeDtypeStruct((nsteps * HB, 128), jnp.int32),
        ],
    )(logits, targets)


CROWS = CHUNK // 128


def _make_stage2(n_rows, f_rows):
    mesh = plsc.VectorSubcoreMesh(core_axis_name="c", subcore_axis_name="s")
    n_per_w = n_rows // NW
    f_per_w = f_rows // NW
    n_chunks = n_per_w // CROWS
    f_chunks = f_per_w // CROWS

    @functools.partial(
        pl.kernel,
        out_type=jax.ShapeDtypeStruct((NW * NSLOT,), jnp.int32),
        mesh=mesh,
        compiler_params=pltpu.CompilerParams(needs_layout_passes=False),
        scratch_types=[
            pltpu.VMEM((CROWS, 128), jnp.int32),
            pltpu.VMEM((CROWS, 128), jnp.int32),
            pltpu.VMEM((NSLOT,), jnp.int32),
            pltpu.SemaphoreType.DMA,
            pltpu.SemaphoreType.DMA,
        ],
    )
    def hist_kernel(idxn_hbm, idxf_hbm, zeros_hbm, out_hbm,
                    buf0, buf1, hist_v, sem0, sem1):
        cid = lax.axis_index("c")
        sid = lax.axis_index("s")
        wid = cid * 16 + sid
        pltpu.sync_copy(zeros_hbm, hist_v)
        bufs = (buf0, buf1)
        sems = (sem0, sem1)

        ones16 = jnp.ones((16,), jnp.int32)

        def process_packed(buf):
            def row_body(r, carry):
                xs = [buf[2 * r + rr, pl.ds(k * 16, 16)]
                      for rr in range(2) for k in range(8)]
                los = [x & 0xFFFF for x in xs]
                his = [lax.shift_right_logical(x, 16) for x in xs]
                for v in los + his:
                    plsc.addupdate_scatter(hist_v, [v], ones16)
                return carry
            lax.fori_loop(0, CROWS // 2, row_body, 0)

        def process_raw(buf):
            def row_body(r, carry):
                xs = [buf[2 * r + rr, pl.ds(k * 16, 16)]
                      for rr in range(2) for k in range(8)]
                for x in xs:
                    plsc.addupdate_scatter(hist_v, [x], ones16)
                return carry
            lax.fori_loop(0, CROWS // 2, row_body, 0)

        def scatter_range(src_hbm, base_row, num_chunks, process):
            for b in range(2):
                pltpu.async_copy(
                    src_hbm.at[pl.ds(base_row + b * CROWS, CROWS)],
                    bufs[b], sems[b])

            def pair_body(i, carry):
                for b in range(2):
                    pltpu.make_async_copy(
                        src_hbm.at[pl.ds(base_row, CROWS)],
                        bufs[b], sems[b]).wait()
                    process(bufs[b])
                    nxt = (2 * i + 2 + b) * CROWS
                    pltpu.async_copy(
                        src_hbm.at[pl.ds(base_row + nxt, CROWS)],
                        bufs[b], sems[b])
                return carry
            lax.fori_loop(0, num_chunks // 2 - 1, pair_body, 0)
            for b in range(2):
                pltpu.make_async_copy(
                    src_hbm.at[pl.ds(base_row, CROWS)],
                    bufs[b], sems[b]).wait()
                process(bufs[b])

        scatter_range(idxn_hbm, wid * n_per_w, n_chunks, process_packed)
        scatter_range(idxf_hbm, wid * f_per_w, f_chunks, process_raw)
        pltpu.sync_copy(hist_v, out_hbm.at[pl.ds(wid * NSLOT, NSLOT)])

    return hist_kernel


ROWS = NCLS * (K // 128)
RPC = K // 128


def _finish_body(nh_ref, fh_ref, out_ref):
    n = nh_ref[0]
    f = fh_ref[0]
    for w in range(1, nh_ref.shape[0]):
        n = n + nh_ref[w]
        f = f + fh_ref[w]
    n = n.astype(jnp.float32)
    f = f.astype(jnp.float32)
    mm = lambda a, b: lax.dot_general(
        a, b, (((1,), (0,)), ((), ())),
        preferred_element_type=jnp.float32)
    ia = lax.broadcasted_iota(jnp.int32, (128, 128), 0)
    ja = lax.broadcasted_iota(jnp.int32, (128, 128), 1)
    tri = (ia <= ja).astype(jnp.float32)
    nrc = mm(n, tri)
    frc = mm(f, tri)
    ib = lax.broadcasted_iota(jnp.int32, (ROWS, ROWS), 0)
    jb = lax.broadcasted_iota(jnp.int32, (ROWS, ROWS), 1)
    same = ib // RPC == jb // RPC
    pre = (same & (jb < ib)).astype(jnp.float32)
    tot = same.astype(jnp.float32)
    rt = jnp.concatenate([nrc[:, 127:128], frc[:, 127:128]], axis=1)
    pref = mm(pre, rt)
    icum = nrc + pref[:, 0:1]
    fcum = frc + pref[:, 1:2]
    G = mm(tot, rt[:, 1:2])
    J = 1.0 - (G - fcum) / jnp.maximum(G + icum - fcum, 1.0)
    present = (G > 0).astype(jnp.float32)
    tot_j = jnp.sum(J * present)
    npres = jnp.sum(present) * (1.0 / RPC)
    loss = ((tot_j - npres) / (K - 1)) / jnp.maximum(npres, 1.0)
    out_ref[...] = jnp.full((1, 1), loss, jnp.float32)


def _finish(nh, fh):
    return pl.pallas_call(
        _finish_body,
        out_shape=jax.ShapeDtypeStruct((1, 1), jnp.float32),
    )(nh, fh)


def kernel(logits, targets):
    B, C, H, W = logits.shape
    idxn2, idxf2 = _stage1(logits.astype(jnp.float32),
                           targets.astype(jnp.int32), B, H, W)
    zeros = jnp.zeros((NSLOT,), jnp.int32)
    hist = _make_stage2(idxn2.shape[0], idxf2.shape[0])(
        idxn2, idxf2, zeros).reshape(NW, NSLOT)
    nh = hist[:, :CK].reshape(NW, ROWS, 128)
    fh = hist[:, CK:2 * CK].reshape(NW, ROWS, 128)
    return _finish(nh, fh).reshape(())


# speedup vs baseline: 203.8003x; 1.1174x over previous
import functools

import jax
import jax.numpy as jnp
from jax import lax
from jax.experimental import pallas as pl
from jax.experimental.pallas import tpu as pltpu
from jax.experimental.pallas import tpu_sc as plsc

IGNORE = 255
NCLS = 19
K = 2048
CK = NCLS * K
HROWS = 616
NSLOT = HROWS * 128
HB = 128
NW = 32
CHUNK = 8192


def _stage1_body(logits_ref, tgt_ref, idx_ref, fg_ref):
    x = logits_ref[0]
    t = tgt_ref[0]
    m = jnp.max(x, axis=0)
    e = jnp.exp(x - m)
    r = (K - 1.0) / jnp.sum(e, axis=0)
    valid = t != IGNORE
    fg_acc = jnp.zeros(t.shape, jnp.int32)
    for c in range(NCLS):
        q = jnp.clip((e[c] * r + 0.5).astype(jnp.int32), 0, K - 1)
        fgc = (t == c) & valid
        rb = jnp.where(fgc, (K - 1) - q, jnp.where(valid, q, K - 1))
        full = rb + (c * K)
        idx_ref[c * (HB // 2):(c + 1) * (HB // 2), :] = (
            full[:HB // 2] | (full[HB // 2:] << 16))
        fg_acc = fg_acc + jnp.where(fgc, (CK + c * K + (K - 1)) - q, 0)
    fg_ref[...] = jnp.where(valid, fg_acc, 2 * CK)


def _stage1(logits, targets, B, H, W):
    grid = (B, H // HB, W // 128)
    nsteps = B * (H // HB) * (W // 128)
    rmap = lambda b, h, w: (b * ((H // HB) * (W // 128)) + h * (W // 128) + w, 0)
    return pl.pallas_call(
        _stage1_body,
        grid=grid,
        in_specs=[
            pl.BlockSpec((1, NCLS, HB, 128), lambda b, h, w: (b, 0, h, w)),
            pl.BlockSpec((1, HB, 128), lambda b, h, w: (b, h, w)),
        ],
        out_specs=[
            pl.BlockSpec((NCLS * (HB // 2), 128), rmap),
            pl.BlockSpec((HB, 128), rmap),
        ],
        out_shape=[
            jax.ShapeDtypeStruct((nsteps * NCLS * (HB // 2), 128), jnp.int32),
            jax.ShapeDtypeStruct((nsteps * HB, 128), jnp.int32),
        ],
    )(logits, targets)


CROWS = CHUNK // 128


def _make_stage2(n_rows, f_rows):
    mesh = plsc.VectorSubcoreMesh(core_axis_name="c", subcore_axis_name="s")
    n_per_w = n_rows // NW
    f_per_w = f_rows // NW
    n_chunks = n_per_w // CROWS
    f_chunks = f_per_w // CROWS

    @functools.partial(
        pl.kernel,
        out_type=jax.ShapeDtypeStruct((NW * HROWS, 128), jnp.int32),
        mesh=mesh,
        compiler_params=pltpu.CompilerParams(needs_layout_passes=False),
        scratch_types=[
            pltpu.VMEM((CROWS, 128), jnp.int32),
            pltpu.VMEM((CROWS, 128), jnp.int32),
            pltpu.VMEM((HROWS, 128), jnp.int32),
            pltpu.SemaphoreType.DMA,
            pltpu.SemaphoreType.DMA,
        ],
    )
    def hist_kernel(idxn_hbm, idxf_hbm, zeros_hbm, out_hbm,
                    buf0, buf1, hist_v, sem0, sem1):
        cid = lax.axis_index("c")
        sid = lax.axis_index("s")
        wid = cid * 16 + sid
        pltpu.sync_copy(zeros_hbm, hist_v)
        bufs = (buf0, buf1)
        sems = (sem0, sem1)

        ones16 = jnp.ones((16,), jnp.int32)

        def scat(v):
            plsc.addupdate_scatter(
                hist_v, [lax.shift_right_logical(v, 7), v & 127], ones16)

        def process_packed(buf):
            def row_body(r, carry):
                xs = [buf[2 * r + rr, pl.ds(k * 16, 16)]
                      for rr in range(2) for k in range(8)]
                los = [x & 0xFFFF for x in xs]
                his = [lax.shift_right_logical(x, 16) for x in xs]
                for v in los + his:
                    scat(v)
                return carry
            lax.fori_loop(0, CROWS // 2, row_body, 0)

        def process_raw(buf):
            def row_body(r, carry):
                xs = [buf[2 * r + rr, pl.ds(k * 16, 16)]
                      for rr in range(2) for k in range(8)]
                for x in xs:
                    scat(x)
                return carry
            lax.fori_loop(0, CROWS // 2, row_body, 0)

        def scatter_range(src_hbm, base_row, num_chunks, process):
            for b in range(2):
                pltpu.async_copy(
                    src_hbm.at[pl.ds(base_row + b * CROWS, CROWS)],
                    bufs[b], sems[b])

            def pair_body(i, carry):
                for b in range(2):
                    pltpu.make_async_copy(
                        src_hbm.at[pl.ds(base_row, CROWS)],
                        bufs[b], sems[b]).wait()
                    process(bufs[b])
                    nxt = (2 * i + 2 + b) * CROWS
                    pltpu.async_copy(
                        src_hbm.at[pl.ds(base_row + nxt, CROWS)],
                        bufs[b], sems[b])
                return carry
            lax.fori_loop(0, num_chunks // 2 - 1, pair_body, 0)
            for b in range(2):
                pltpu.make_async_copy(
                    src_hbm.at[pl.ds(base_row, CROWS)],
                    bufs[b], sems[b]).wait()
                process(bufs[b])

        scatter_range(idxn_hbm, wid * n_per_w, n_chunks, process_packed)
        scatter_range(idxf_hbm, wid * f_per_w, f_chunks, process_raw)
        pltpu.sync_copy(hist_v, out_hbm.at[pl.ds(wid * HROWS, HROWS), :])

    return hist_kernel


ROWS = NCLS * (K // 128)
RPC = K // 128


def _finish_body(hist_ref, out_ref):
    n = hist_ref[0, 0:ROWS]
    f = hist_ref[0, ROWS:2 * ROWS]
    for w in range(1, hist_ref.shape[0]):
        n = n + hist_ref[w, 0:ROWS]
        f = f + hist_ref[w, ROWS:2 * ROWS]
    n = n.astype(jnp.float32)
    f = f.astype(jnp.float32)
    mm = lambda a, b: lax.dot_general(
        a, b, (((1,), (0,)), ((), ())),
        preferred_element_type=jnp.float32)
    ia = lax.broadcasted_iota(jnp.int32, (128, 128), 0)
    ja = lax.broadcasted_iota(jnp.int32, (128, 128), 1)
    tri = (ia <= ja).astype(jnp.float32)
    nrc = mm(n, tri)
    frc = mm(f, tri)
    ib = lax.broadcasted_iota(jnp.int32, (ROWS, ROWS), 0)
    jb = lax.broadcasted_iota(jnp.int32, (ROWS, ROWS), 1)
    same = ib // RPC == jb // RPC
    pre = (same & (jb < ib)).astype(jnp.float32)
    tot = same.astype(jnp.float32)
    rt = jnp.concatenate([nrc[:, 127:128], frc[:, 127:128]], axis=1)
    pref = mm(pre, rt)
    icum = nrc + pref[:, 0:1]
    fcum = frc + pref[:, 1:2]
    G = mm(tot, rt[:, 1:2])
    J = 1.0 - (G - fcum) / jnp.maximum(G + icum - fcum, 1.0)
    present = (G > 0).astype(jnp.float32)
    tot_j = jnp.sum(J * present)
    npres = jnp.sum(present) * (1.0 / RPC)
    loss = ((tot_j - npres) / (K - 1)) / jnp.maximum(npres, 1.0)
    out_ref[...] = jnp.full((1, 1), loss, jnp.float32)


def _finish(hist):
    return pl.pallas_call(
        _finish_body,
        out_shape=jax.ShapeDtypeStruct((1, 1), jnp.float32),
    )(hist)


def kernel(logits, targets):
    B, C, H, W = logits.shape
    idxn2, idxf2 = _stage1(logits.astype(jnp.float32),
                           targets.astype(jnp.int32), B, H, W)
    zeros = jnp.zeros((HROWS, 128), jnp.int32)
    hist = _make_stage2(idxn2.shape[0], idxf2.shape[0])(
        idxn2, idxf2, zeros).reshape(NW, HROWS, 128)
    return _finish(hist).reshape(())


# speedup vs baseline: 204.1850x; 1.0019x over previous
import functools

import jax
import jax.numpy as jnp
from jax import lax
from jax.experimental import pallas as pl
from jax.experimental.pallas import tpu as pltpu
from jax.experimental.pallas import tpu_sc as plsc

IGNORE = 255
NCLS = 19
K = 2048
CK = NCLS * K
HROWS = 616
NSLOT = HROWS * 128
HB = 128
NW = 32
CHUNK = 8192


def _stage1_body(logits_ref, tgt_ref, idx_ref, fg_ref):
    x = logits_ref[0]
    t = tgt_ref[0]
    m = jnp.max(x, axis=0)
    e = jnp.exp(x - m)
    r = (K - 1.0) / jnp.sum(e, axis=0)
    valid = t != IGNORE
    fg_acc = jnp.zeros(t.shape, jnp.int32)
    for c in range(NCLS):
        q = jnp.clip((e[c] * r + 0.5).astype(jnp.int32), 0, K - 1)
        fgc = (t == c) & valid
        rb = jnp.where(fgc, (K - 1) - q, jnp.where(valid, q, K - 1))
        full = rb + (c * K)
        idx_ref[c * (HB // 2):(c + 1) * (HB // 2), :] = (
            full[:HB // 2] | (full[HB // 2:] << 16))
        fg_acc = fg_acc + jnp.where(fgc, (CK + c * K + (K - 1)) - q, 0)
    fg_ref[...] = jnp.where(valid, fg_acc, 2 * CK)


def _stage1(logits, targets, b0, nb, H, W):
    grid = (nb, H // HB, W // 128)
    nsteps = nb * (H // HB) * (W // 128)
    rmap = lambda b, h, w: (b * ((H // HB) * (W // 128)) + h * (W // 128) + w, 0)
    return pl.pallas_call(
        _stage1_body,
        grid=grid,
        in_specs=[
            pl.BlockSpec((1, NCLS, HB, 128), lambda b, h, w: (b0 + b, 0, h, w)),
            pl.BlockSpec((1, HB, 128), lambda b, h, w: (b0 + b, h, w)),
        ],
        out_specs=[
            pl.BlockSpec((NCLS * (HB // 2), 128), rmap),
            pl.BlockSpec((HB, 128), rmap),
        ],
        out_shape=[
            jax.ShapeDtypeStruct((nsteps * NCLS * (HB // 2), 128), jnp.int32),
            jax.ShapeDtypeStruct((nsteps * HB, 128), jnp.int32),
        ],
    )(logits, targets)


CROWS = 32


def _make_stage2(n_rows, f_rows):
    mesh = plsc.VectorSubcoreMesh(core_axis_name="c", subcore_axis_name="s")
    n_per_w = n_rows // NW
    f_per_w = f_rows // NW
    n_chunks = n_per_w // CROWS
    f_chunks = f_per_w // CROWS

    @functools.partial(
        pl.kernel,
        out_type=jax.ShapeDtypeStruct((NW * HROWS, 128), jnp.int32),
        mesh=mesh,
        compiler_params=pltpu.CompilerParams(needs_layout_passes=False),
        scratch_types=[
            pltpu.VMEM((CROWS, 128), jnp.int32),
            pltpu.VMEM((CROWS, 128), jnp.int32),
            pltpu.VMEM((HROWS, 128), jnp.int32),
            pltpu.SemaphoreType.DMA,
            pltpu.SemaphoreType.DMA,
        ],
    )
    def hist_kernel(idxn_hbm, idxf_hbm, zeros_hbm, out_hbm,
                    buf0, buf1, hist_v, sem0, sem1):
        cid = lax.axis_index("c")
        sid = lax.axis_index("s")
        wid = cid * 16 + sid
        pltpu.sync_copy(zeros_hbm, hist_v)
        bufs = (buf0, buf1)
        sems = (sem0, sem1)

        ones16 = jnp.ones((16,), jnp.int32)

        def scat(v):
            plsc.addupdate_scatter(
                hist_v, [lax.shift_right_logical(v, 7), v & 127], ones16)

        def process_packed(buf):
            def row_body(r, carry):
                xs = [buf[2 * r + rr, pl.ds(k * 16, 16)]
                      for rr in range(2) for k in range(8)]
                los = [x & 0xFFFF for x in xs]
                his = [lax.shift_right_logical(x, 16) for x in xs]
                for v in los + his:
                    scat(v)
                return carry
            lax.fori_loop(0, CROWS // 2, row_body, 0)

        def process_raw(buf):
            def row_body(r, carry):
                xs = [buf[2 * r + rr, pl.ds(k * 16, 16)]
                      for rr in range(2) for k in range(8)]
                for x in xs:
                    scat(x)
                return carry
            lax.fori_loop(0, CROWS // 2, row_body, 0)

        def scatter_range(src_hbm, base_row, num_chunks, process):
            for b in range(2):
                pltpu.async_copy(
                    src_hbm.at[pl.ds(base_row + b * CROWS, CROWS)],
                    bufs[b], sems[b])

            def pair_body(i, carry):
                for b in range(2):
                    pltpu.make_async_copy(
                        src_hbm.at[pl.ds(base_row, CROWS)],
                        bufs[b], sems[b]).wait()
                    process(bufs[b])
                    nxt = (2 * i + 2 + b) * CROWS
                    pltpu.async_copy(
                        src_hbm.at[pl.ds(base_row + nxt, CROWS)],
                        bufs[b], sems[b])
                return carry
            lax.fori_loop(0, num_chunks // 2 - 1, pair_body, 0)
            for b in range(2):
                pltpu.make_async_copy(
                    src_hbm.at[pl.ds(base_row, CROWS)],
                    bufs[b], sems[b]).wait()
                process(bufs[b])

        scatter_range(idxn_hbm, wid * n_per_w, n_chunks, process_packed)
        scatter_range(idxf_hbm, wid * f_per_w, f_chunks, process_raw)
        pltpu.sync_copy(hist_v, out_hbm.at[pl.ds(wid * HROWS, HROWS), :])

    return hist_kernel


ROWS = NCLS * (K // 128)
RPC = K // 128


def _finish_body(*refs):
    hist_refs, out_ref = refs[:-1], refs[-1]
    n = None
    f = None
    for hist_ref in hist_refs:
        for w in range(hist_ref.shape[0]):
            nw = hist_ref[w, 0:ROWS]
            fw = hist_ref[w, ROWS:2 * ROWS]
            n = nw if n is None else n + nw
            f = fw if f is None else f + fw
    n = n.astype(jnp.float32)
    f = f.astype(jnp.float32)
    mm = lambda a, b: lax.dot_general(
        a, b, (((1,), (0,)), ((), ())),
        preferred_element_type=jnp.float32)
    ia = lax.broadcasted_iota(jnp.int32, (128, 128), 0)
    ja = lax.broadcasted_iota(jnp.int32, (128, 128), 1)
    tri = (ia <= ja).astype(jnp.float32)
    nrc = mm(n, tri)
    frc = mm(f, tri)
    ib = lax.broadcasted_iota(jnp.int32, (ROWS, ROWS), 0)
    jb = lax.broadcasted_iota(jnp.int32, (ROWS, ROWS), 1)
    same = ib // RPC == jb // RPC
    pre = (same & (jb < ib)).astype(jnp.float32)
    tot = same.astype(jnp.float32)
    rt = jnp.concatenate([nrc[:, 127:128], frc[:, 127:128]], axis=1)
    pref = mm(pre, rt)
    icum = nrc + pref[:, 0:1]
    fcum = frc + pref[:, 1:2]
    G = mm(tot, rt[:, 1:2])
    J = 1.0 - (G - fcum) / jnp.maximum(G + icum - fcum, 1.0)
    present = (G > 0).astype(jnp.float32)
    tot_j = jnp.sum(J * present)
    npres = jnp.sum(present) * (1.0 / RPC)
    loss = ((tot_j - npres) / (K - 1)) / jnp.maximum(npres, 1.0)
    out_ref[...] = jnp.full((1, 1), loss, jnp.float32)


def _finish(hists):
    return pl.pallas_call(
        _finish_body,
        out_shape=jax.ShapeDtypeStruct((1, 1), jnp.float32),
    )(*hists)


NSPLIT = 2


def kernel(logits, targets):
    B, C, H, W = logits.shape
    lg = logits.astype(jnp.float32)
    tg = targets.astype(jnp.int32)
    nb = B // NSPLIT
    zeros = jnp.zeros((HROWS, 128), jnp.int32)
    parts = [_stage1(lg, tg, s * nb, nb, H, W) for s in range(NSPLIT)]
    s2 = _make_stage2(parts[0][0].shape[0], parts[0][1].shape[0])
    hists = [s2(idxn2, idxf2, zeros).reshape(NW, HROWS, 128)
             for idxn2, idxf2 in parts]
    return _finish(hists).reshape(())
